# baseline (speedup 1.0000x reference)
import math

import jax
import jax.numpy as jnp
from jax import lax
from jax.experimental import pallas as pl
from jax.experimental.pallas import tpu as pltpu

_CHUNK = 1024
_NBUF = 4


def _pe_add_manual(x_hbm, mask_hbm, pe_hbm, out_hbm,
                   pe_v, mask_v, xin, xout, in_sems, out_sems):
    R, D = x_hbm.shape
    S = pe_v.shape[0]
    NCHUNKS = R // _CHUNK
    scale = math.sqrt(D)

    def in_copy(c, k):
        return pltpu.make_async_copy(
            x_hbm.at[pl.ds(c * _CHUNK, _CHUNK), :], xin.at[k], in_sems.at[k])

    def mask_copy(c, k):
        return pltpu.make_async_copy(
            mask_hbm.at[pl.ds(c * _CHUNK, _CHUNK), :], mask_v.at[k],
            in_sems.at[k])

    def out_copy(c, k):
        return pltpu.make_async_copy(
            xout.at[k], out_hbm.at[pl.ds(c * _CHUNK, _CHUNK), :], out_sems.at[k])

    pe_copy = pltpu.make_async_copy(
        pe_hbm.at[pl.ds(0, S), :], pe_v, out_sems.at[0])
    pe_copy.start()
    for k in range(_NBUF):
        in_copy(k, k).start()
        mask_copy(k, k).start()
    pe_copy.wait()

    pe_rows = pe_v[...]

    def step(m, _):
        for k in range(_NBUF):
            c = m * _NBUF + k
            in_copy(c, k).wait()
            mask_copy(c, k).wait()

            @pl.when(c >= _NBUF)
            def _wait_prev_out():
                out_copy(c - _NBUF, k).wait()

            for j in range(_CHUNK // S):
                sl = pl.ds(j * S, S)
                mrows = mask_v[k, sl, :]
                keep = (mrows != 1).astype(jnp.float32)
                xout[k, sl, :] = scale * xin[k, sl, :] + keep * pe_rows

            out_copy(c, k).start()

            @pl.when(c + _NBUF < NCHUNKS)
            def _next_in():
                in_copy(c + _NBUF, k).start()
                mask_copy(c + _NBUF, k).start()
        return 0

    lax.fori_loop(0, NCHUNKS // _NBUF, step, 0)
    for k in range(_NBUF):
        out_copy(NCHUNKS - _NBUF + k, k).wait()


def kernel(x, mask, pe):
    B, S, D = x.shape
    x2 = x.reshape(B * S, D)
    mask2 = mask.reshape(B * S, 1)

    out = pl.pallas_call(
        _pe_add_manual,
        in_specs=[
            pl.BlockSpec(memory_space=pl.ANY),
            pl.BlockSpec(memory_space=pl.ANY),
            pl.BlockSpec(memory_space=pl.ANY),
        ],
        out_specs=pl.BlockSpec(memory_space=pl.ANY),
        out_shape=jax.ShapeDtypeStruct((B * S, D), x.dtype),
        scratch_shapes=[
            pltpu.VMEM((S, D), jnp.float32),
            pltpu.VMEM((_NBUF, _CHUNK, 1), jnp.int32),
            pltpu.VMEM((_NBUF, _CHUNK, D), jnp.float32),
            pltpu.VMEM((_NBUF, _CHUNK, D), jnp.float32),
            pltpu.SemaphoreType.DMA((_NBUF,)),
            pltpu.SemaphoreType.DMA((_NBUF,)),
        ],
    )(x2, mask2, pe)
    return out.reshape(B, S, D)


# speedup vs baseline: 1.0158x; 1.0158x over previous
import math

import jax
import jax.numpy as jnp
from jax import lax
from jax.experimental import pallas as pl
from jax.experimental.pallas import tpu as pltpu

_CHUNK = 1024
_NBUF = 4


def _pe_add_manual(x_hbm, mask_hbm, pe_hbm, out_hbm,
                   pe_v, mask_v, xin, xout, in_sems, out_sems):
    R, D = x_hbm.shape
    S = pe_v.shape[0]
    NCHUNKS = R // _CHUNK
    scale = math.sqrt(D)

    def in_copy(c, k):
        return pltpu.make_async_copy(
            x_hbm.at[pl.ds(c * _CHUNK, _CHUNK), :], xin.at[k], in_sems.at[k])

    def out_copy(c, k):
        return pltpu.make_async_copy(
            xout.at[k], out_hbm.at[pl.ds(c * _CHUNK, _CHUNK), :], out_sems.at[k])

    pe_copy = pltpu.make_async_copy(
        pe_hbm.at[pl.ds(0, S), :], pe_v, out_sems.at[0])
    mask_copy = pltpu.make_async_copy(mask_hbm, mask_v, out_sems.at[1])
    pe_copy.start()
    mask_copy.start()
    for k in range(_NBUF):
        in_copy(k, k).start()
    pe_copy.wait()
    mask_copy.wait()

    pe_rows = pe_v[...]

    def step(m, _):
        for k in range(_NBUF):
            c = m * _NBUF + k
            in_copy(c, k).wait()

            @pl.when(c >= _NBUF)
            def _wait_prev_out():
                out_copy(c - _NBUF, k).wait()

            for j in range(_CHUNK // S):
                sl = pl.ds(j * S, S)
                mrows = mask_v[pl.ds(c * _CHUNK + j * S, S), :]
                keep = (mrows != 1).astype(jnp.float32)
                xout[k, sl, :] = scale * xin[k, sl, :] + keep * pe_rows

            out_copy(c, k).start()

            @pl.when(c + _NBUF < NCHUNKS)
            def _next_in():
                in_copy(c + _NBUF, k).start()
        return 0

    lax.fori_loop(0, NCHUNKS // _NBUF, step, 0)
    for k in range(_NBUF):
        out_copy(NCHUNKS - _NBUF + k, k).wait()


def kernel(x, mask, pe):
    B, S, D = x.shape
    x2 = x.reshape(B * S, D)
    mask2 = mask.reshape(B * S, 1)

    out = pl.pallas_call(
        _pe_add_manual,
        in_specs=[
            pl.BlockSpec(memory_space=pl.ANY),
            pl.BlockSpec(memory_space=pl.ANY),
            pl.BlockSpec(memory_space=pl.ANY),
        ],
        out_specs=pl.BlockSpec(memory_space=pl.ANY),
        out_shape=jax.ShapeDtypeStruct((B * S, D), x.dtype),
        scratch_shapes=[
            pltpu.VMEM((S, D), jnp.float32),
            pltpu.VMEM((B * S, 1), jnp.int32),
            pltpu.VMEM((_NBUF, _CHUNK, D), jnp.float32),
            pltpu.VMEM((_NBUF, _CHUNK, D), jnp.float32),
            pltpu.SemaphoreType.DMA((_NBUF,)),
            pltpu.SemaphoreType.DMA((_NBUF,)),
        ],
    )(x2, mask2, pe)
    return out.reshape(B, S, D)
